# Initial kernel scaffold; baseline (speedup 1.0000x reference)
import jax
import jax.numpy as jnp
from jax.experimental import pallas as pl


def kernel(x, fixed_edge_index, fixed_edge_attr, W, mix_logit):
    raise NotImplementedError("write your pallas kernel here")


# speedup vs baseline: 20.0936x; 20.0936x over previous
import functools

import jax
import jax.numpy as jnp
from jax.experimental import pallas as pl
from jax.experimental.pallas import tpu as pltpu

TOPK = 16


def _embed_kernel(x_ref, w_ref, e_ref):
    xm = jnp.mean(x_ref[...], axis=1)
    e = jax.lax.dot_general(
        xm, w_ref[...], (((1,), (1,)), ((), ())),
        preferred_element_type=jnp.float32)
    nrm = jnp.sqrt(jnp.sum(e * e, axis=1, keepdims=True))
    e_ref[...] = e / jnp.maximum(nrm, 1e-12)


def _topk_kernel(n_real, k, e_blk_ref, e_all_ref, mix_ref, vals_ref, idx_ref):
    br = e_blk_ref.shape[0]
    npad = e_all_ref.shape[0]
    i = pl.program_id(0)
    sim = jax.lax.dot_general(
        e_blk_ref[...], e_all_ref[...], (((1,), (1,)), ((), ())),
        preferred_element_type=jnp.float32)
    col = jax.lax.broadcasted_iota(jnp.int32, (br, npad), 1)
    row = i * br + jax.lax.broadcasted_iota(jnp.int32, (br, npad), 0)
    neg = jnp.float32(-jnp.inf)
    big = jnp.int32(2**31 - 1)
    sim = jnp.where((col >= n_real) | (col == row), neg, sim)
    vals = jnp.zeros((br, k), jnp.float32)
    idx = jnp.zeros((br, k), jnp.int32)
    lane = jax.lax.broadcasted_iota(jnp.int32, (br, k), 1)
    for t in range(k):
        m = jnp.max(sim, axis=1, keepdims=True)
        cand = jnp.where(sim == m, col, big)
        a = jnp.min(cand, axis=1, keepdims=True)
        vals = jnp.where(lane == t, m, vals)
        idx = jnp.where(lane == t, a, idx)
        sim = jnp.where(cand == a, neg, sim)
    alpha = 1.0 / (1.0 + jnp.exp(-mix_ref[0]))
    vals_ref[...] = vals * alpha
    idx_ref[...] = idx


def _scale_kernel(attr_ref, mix_ref, out_ref):
    alpha = 1.0 / (1.0 + jnp.exp(-mix_ref[0]))
    out_ref[...] = attr_ref[...] * (1.0 - alpha)


def _largest_divisor(n, cap):
    for d in range(min(n, cap), 0, -1):
        if n % d == 0 and (d % 8 == 0 or d == n):
            return d
    return n


def kernel(x, fixed_edge_index, fixed_edge_attr, W, mix_logit):
    n, t, h = x.shape
    d = W.shape[0]
    k = min(TOPK, n - 1)
    mix1 = jnp.reshape(mix_logit, (1,))

    br_a = _largest_divisor(n, 500)
    e = pl.pallas_call(
        _embed_kernel,
        grid=(n // br_a,),
        in_specs=[
            pl.BlockSpec((br_a, t, h), lambda i: (i, 0, 0)),
            pl.BlockSpec((d, h), lambda i: (0, 0)),
        ],
        out_specs=pl.BlockSpec((br_a, d), lambda i: (i, 0)),
        out_shape=jax.ShapeDtypeStruct((n, d), jnp.float32),
    )(x, W)

    br = 128
    npad = ((n + br - 1) // br) * br
    e_pad = jnp.pad(e, ((0, npad - n), (0, 0)))
    vals, idx = pl.pallas_call(
        functools.partial(_topk_kernel, n, k),
        grid=(npad // br,),
        in_specs=[
            pl.BlockSpec((br, d), lambda i: (i, 0)),
            pl.BlockSpec((npad, d), lambda i: (0, 0)),
            pl.BlockSpec(memory_space=pltpu.SMEM),
        ],
        out_specs=[
            pl.BlockSpec((br, k), lambda i: (i, 0)),
            pl.BlockSpec((br, k), lambda i: (i, 0)),
        ],
        out_shape=[
            jax.ShapeDtypeStruct((npad, k), jnp.float32),
            jax.ShapeDtypeStruct((npad, k), jnp.int32),
        ],
    )(e_pad, e_pad, mix1)
    vals = vals[:n]
    idx = idx[:n]

    e_fixed = fixed_edge_attr.shape[0]
    ep = ((e_fixed + 1023) // 1024) * 1024
    fa = jnp.pad(fixed_edge_attr.reshape(-1), (0, ep - e_fixed))
    fa = fa.reshape(ep // 128, 128)
    fattr = pl.pallas_call(
        _scale_kernel,
        in_specs=[
            pl.BlockSpec(fa.shape, lambda: (0, 0)),
            pl.BlockSpec(memory_space=pltpu.SMEM),
        ],
        out_specs=pl.BlockSpec(fa.shape, lambda: (0, 0)),
        out_shape=jax.ShapeDtypeStruct(fa.shape, jnp.float32),
    )(fa, mix1)
    fattr = fattr.reshape(-1)[:e_fixed].reshape(-1, 1)

    src = jnp.repeat(jnp.arange(n, dtype=jnp.int32), k)
    dyn_edge_index = jnp.stack([src, idx.reshape(-1)], axis=0)
    combined_edge_index = jnp.concatenate([fixed_edge_index, dyn_edge_index], axis=1)
    combined_edge_attr = jnp.concatenate([fattr, vals.reshape(-1, 1)], axis=0)
    return combined_edge_index, combined_edge_attr


# speedup vs baseline: 24.1237x; 1.2006x over previous
import functools

import jax
import jax.numpy as jnp
from jax.experimental import pallas as pl
from jax.experimental.pallas import tpu as pltpu

TOPK = 16


def _embed_kernel(x_ref, w_ref, e_ref):
    xm = jnp.mean(x_ref[...], axis=1)
    e = jax.lax.dot_general(
        xm, w_ref[...], (((1,), (1,)), ((), ())),
        preferred_element_type=jnp.float32)
    nrm = jnp.sqrt(jnp.sum(e * e, axis=1, keepdims=True))
    e_ref[...] = e / jnp.maximum(nrm, 1e-12)


def _topk_kernel(n_real, k, e_blk_ref, e_all_ref, mix_ref, vals_ref, idx_ref):
    br = e_blk_ref.shape[0]
    npad = e_all_ref.shape[0]
    i = pl.program_id(0)
    sim = jax.lax.dot_general(
        e_blk_ref[...], e_all_ref[...], (((1,), (1,)), ((), ())),
        preferred_element_type=jnp.float32)
    col = jax.lax.broadcasted_iota(jnp.int32, (br, npad), 1)
    row = i * br + jax.lax.broadcasted_iota(jnp.int32, (br, npad), 0)
    neg = jnp.float32(-jnp.inf)
    colf = col.astype(jnp.float32)
    bigf = jnp.float32(3e38)
    sim = jnp.where((col >= n_real) | (col == row), neg, sim)
    vals = jnp.zeros((br, k), jnp.float32)
    idxf = jnp.zeros((br, k), jnp.float32)
    lane = jax.lax.broadcasted_iota(jnp.int32, (br, k), 1)
    for t in range(k):
        m = jnp.max(sim, axis=1, keepdims=True)
        a = jnp.min(jnp.where(sim == m, colf, bigf), axis=1, keepdims=True)
        vals = jnp.where(lane == t, m, vals)
        idxf = jnp.where(lane == t, a, idxf)
        sim = jnp.where(colf == a, neg, sim)
    alpha = 1.0 / (1.0 + jnp.exp(-mix_ref[0]))
    vals_ref[...] = vals * alpha
    idx_ref[...] = idxf.astype(jnp.int32)


def _scale_kernel(attr_ref, mix_ref, out_ref):
    alpha = 1.0 / (1.0 + jnp.exp(-mix_ref[0]))
    out_ref[...] = attr_ref[...] * (1.0 - alpha)


def _largest_divisor(n, cap):
    for d in range(min(n, cap), 0, -1):
        if n % d == 0 and (d % 8 == 0 or d == n):
            return d
    return n


def kernel(x, fixed_edge_index, fixed_edge_attr, W, mix_logit):
    n, t, h = x.shape
    d = W.shape[0]
    k = min(TOPK, n - 1)
    mix1 = jnp.reshape(mix_logit, (1,))

    br_a = _largest_divisor(n, 500)
    e = pl.pallas_call(
        _embed_kernel,
        grid=(n // br_a,),
        in_specs=[
            pl.BlockSpec((br_a, t, h), lambda i: (i, 0, 0)),
            pl.BlockSpec((d, h), lambda i: (0, 0)),
        ],
        out_specs=pl.BlockSpec((br_a, d), lambda i: (i, 0)),
        out_shape=jax.ShapeDtypeStruct((n, d), jnp.float32),
    )(x, W)

    br = 128
    npad = ((n + br - 1) // br) * br
    e_pad = jnp.pad(e, ((0, npad - n), (0, 0)))
    vals, idx = pl.pallas_call(
        functools.partial(_topk_kernel, n, k),
        grid=(npad // br,),
        in_specs=[
            pl.BlockSpec((br, d), lambda i: (i, 0)),
            pl.BlockSpec((npad, d), lambda i: (0, 0)),
            pl.BlockSpec(memory_space=pltpu.SMEM),
        ],
        out_specs=[
            pl.BlockSpec((br, k), lambda i: (i, 0)),
            pl.BlockSpec((br, k), lambda i: (i, 0)),
        ],
        out_shape=[
            jax.ShapeDtypeStruct((npad, k), jnp.float32),
            jax.ShapeDtypeStruct((npad, k), jnp.int32),
        ],
    )(e_pad, e_pad, mix1)
    vals = vals[:n]
    idx = idx[:n]

    e_fixed = fixed_edge_attr.shape[0]
    ep = ((e_fixed + 1023) // 1024) * 1024
    fa = jnp.pad(fixed_edge_attr.reshape(-1), (0, ep - e_fixed))
    fa = fa.reshape(ep // 128, 128)
    fattr = pl.pallas_call(
        _scale_kernel,
        in_specs=[
            pl.BlockSpec(fa.shape, lambda: (0, 0)),
            pl.BlockSpec(memory_space=pltpu.SMEM),
        ],
        out_specs=pl.BlockSpec(fa.shape, lambda: (0, 0)),
        out_shape=jax.ShapeDtypeStruct(fa.shape, jnp.float32),
    )(fa, mix1)
    fattr = fattr.reshape(-1)[:e_fixed].reshape(-1, 1)

    src = jnp.repeat(jnp.arange(n, dtype=jnp.int32), k)
    dyn_edge_index = jnp.stack([src, idx.reshape(-1)], axis=0)
    combined_edge_index = jnp.concatenate([fixed_edge_index, dyn_edge_index], axis=1)
    combined_edge_attr = jnp.concatenate([fattr, vals.reshape(-1, 1)], axis=0)
    return combined_edge_index, combined_edge_attr
